# baseline (speedup 1.0000x reference)
import functools

import jax
import jax.numpy as jnp
import numpy as np
from jax import lax
from jax.experimental import pallas as pl
from jax.experimental.pallas import tpu as pltpu
from jax.experimental.pallas import tpu_sc as plsc

HEADS = 8
DH = 16
C = 128
NCORE = 2
NSUB = 16
NW = NCORE * NSUB
LANES = 16


def _block_diag(W):
    return jax.scipy.linalg.block_diag(*[W[g] for g in range(W.shape[0])])


def _dense1(xp, W1, b1, Wqb, bqb, Wkb, bkb, Wvb, bvb):
    NT = xp.shape[0]
    RB = 1024

    def body(x_ref, w1, b1r, wq, bqr, wk, bkr, wv, bvr, h_ref, q_ref, k_ref, v_ref):
        h = jnp.maximum(
            jnp.dot(x_ref[...], w1[...], preferred_element_type=jnp.float32) + b1r[...], 0.0)
        h_ref[...] = h
        q_ref[...] = jnp.dot(h, wq[...], preferred_element_type=jnp.float32) + bqr[...]
        k_ref[...] = jnp.dot(h, wk[...], preferred_element_type=jnp.float32) + bkr[...]
        v_ref[...] = jnp.dot(h, wv[...], preferred_element_type=jnp.float32) + bvr[...]

    bs_r = pl.BlockSpec((RB, C), lambda i: (i, 0))
    bs_r2 = pl.BlockSpec((RB, 2 * C), lambda i: (i, 0))
    bs_w = pl.BlockSpec((C, C), lambda i: (0, 0))
    bs_b = pl.BlockSpec((1, C), lambda i: (0, 0))
    out = jax.ShapeDtypeStruct((NT, C), jnp.float32)
    return pl.pallas_call(
        body,
        grid=(NT // RB,),
        in_specs=[bs_r, bs_w, bs_b, bs_w, bs_b, bs_w, bs_b, bs_w, bs_b],
        out_specs=[bs_r, bs_r, bs_r, bs_r],
        out_shape=[out, out, out, out],
    )(xp, W1, b1, Wqb, bqb, Wkb, bkb, Wvb, bvb)


def _dense2(magg, h, Wqb, bqb, Wkb, bkb, Wvb, bvb):
    NT = h.shape[0]
    RB = 1024

    def body(m_ref, h_ref, wq, bqr, wk, bkr, wv, bvr, q_ref, k_ref, v_ref):
        m = m_ref[...]
        h0 = h_ref[...]
        h1 = jnp.maximum(m[0] + m[1], 0.0)
        q_ref[...] = jnp.dot(h1, wq[...], preferred_element_type=jnp.float32) + bqr[...]
        k_ref[:, :C] = jnp.dot(h0, wk[...], preferred_element_type=jnp.float32) + bkr[...]
        k_ref[:, C:] = jnp.dot(h1, wk[...], preferred_element_type=jnp.float32) + bkr[...]
        v_ref[:, :C] = jnp.dot(h0, wv[...], preferred_element_type=jnp.float32) + bvr[...]
        v_ref[:, C:] = jnp.dot(h1, wv[...], preferred_element_type=jnp.float32) + bvr[...]

    bs_m = pl.BlockSpec((2, RB, C), lambda i: (0, i, 0))
    bs_r = pl.BlockSpec((RB, C), lambda i: (i, 0))
    bs_r2 = pl.BlockSpec((RB, 2 * C), lambda i: (i, 0))
    bs_w = pl.BlockSpec((C, C), lambda i: (0, 0))
    bs_b = pl.BlockSpec((1, C), lambda i: (0, 0))
    return pl.pallas_call(
        body,
        grid=(NT // RB,),
        in_specs=[bs_m, bs_r, bs_w, bs_b, bs_w, bs_b, bs_w, bs_b],
        out_specs=[bs_r, bs_r2, bs_r2],
        out_shape=[
            jax.ShapeDtypeStruct((NT, C), jnp.float32),
            jax.ShapeDtypeStruct((NT, 2 * C), jnp.float32),
            jax.ShapeDtypeStruct((NT, 2 * C), jnp.float32),
        ],
    )(magg, h, Wqb, bqb, Wkb, bkb, Wvb, bvb)


def _dense3(magg, W2p, b2p):
    NT = magg.shape[1]
    RB = 1024

    def body(m_ref, w2, b2r, o_ref):
        m = m_ref[...]
        x2 = jnp.maximum(m[0] + m[1], 0.0)
        l = jnp.dot(x2, w2[...], preferred_element_type=jnp.float32) + b2r[...]
        mx = jnp.max(l, axis=1, keepdims=True)
        e = jnp.exp(l - mx)
        o_ref[...] = l - mx - jnp.log(jnp.sum(e, axis=1, keepdims=True))

    bs_m = pl.BlockSpec((2, RB, C), lambda i: (0, i, 0))
    bs_r = pl.BlockSpec((RB, C), lambda i: (i, 0))
    bs_w = pl.BlockSpec((C, C), lambda i: (0, 0))
    bs_b = pl.BlockSpec((1, C), lambda i: (0, 0))
    return pl.pallas_call(
        body,
        grid=(NT // RB,),
        in_specs=[bs_m, bs_w, bs_b],
        out_specs=bs_r,
        out_shape=jax.ShapeDtypeStruct((NT, C), jnp.float32),
    )(magg, W2p, b2p)


def _quake_rsqrt(v):
    i = plsc.bitcast(v, jnp.int32)
    i = jnp.int32(0x5F3759DF) - (i >> 1)
    y = plsc.bitcast(i, jnp.float32)
    for _ in range(3):
        y = y * (1.5 - 0.5 * v * y * y)
    return y


def _make_edge_kernel(L, N, NT, EP, first):
    CD = L * C
    EPW = EP // NW
    EPS = EP // NSUB
    NBLK = 3
    BLK = EPW // NBLK
    NGB = BLK // LANES
    ROWS_S = NT // NSUB

    mesh = plsc.VectorSubcoreMesh(core_axis_name="c", subcore_axis_name="s")

    out_type = [jax.ShapeDtypeStruct((NCORE, NT, C), jnp.float32)]
    if first:
        out_type.append(jax.ShapeDtypeStruct((EP,), jnp.float32))

    scratch = [
        pltpu.VMEM((BLK,), jnp.int32),
        pltpu.VMEM((BLK,), jnp.int32),
        pltpu.VMEM((BLK,), jnp.float32),
        pltpu.VMEM((2 * LANES, C), jnp.float32),
        pltpu.VMEM((2 * LANES, CD), jnp.float32),
        pltpu.VMEM((2 * LANES, CD), jnp.float32),
        pltpu.VMEM((LANES, C), jnp.float32),
    ]
    if first:
        scratch += [
            pltpu.VMEM((LANES, LANES), jnp.float32),
            pltpu.VMEM((64, 16), jnp.float32),
            pltpu.VMEM((16, 16), jnp.float32),
            pltpu.VMEM((ROWS_S,), jnp.float32),
            pltpu.VMEM((NT,), jnp.float32),
            pltpu.VMEM_SHARED((NT, 16), jnp.float32),
            pltpu.VMEM_SHARED((NT,), jnp.float32),
        ]
    scratch += [pltpu.VMEM_SHARED((NT, C), jnp.float32)]
    scratch += [pltpu.SemaphoreType.DMA] * 6

    @functools.partial(
        pl.kernel,
        out_type=tuple(out_type) if first else out_type[0],
        mesh=mesh,
        compiler_params=pltpu.CompilerParams(
            needs_layout_passes=False, use_tc_tiling_on_sc=False),
        scratch_types=scratch,
    )
    def edge_kernel(*refs):
        it = iter(refs)
        qt, kt, vt, rowh, colh = (next(it) for _ in range(5))
        normh = None if first else next(it)
        out = next(it)
        normo = next(it) if first else None
        rowv, colv, normv, qb, kb, vb, mb = (next(it) for _ in range(7))
        if first:
            oneb, zb16, degs, diss, disv, acc16, dis_sh = (
                next(it) for _ in range(7))
        acc = next(it)
        sems = [next(it) for _ in range(6)]
        semq, semk, semv = sems[0:2], sems[2:4], sems[4:6]

        cid = lax.axis_index("c")
        sid = lax.axis_index("s")
        wid = sid * NCORE + cid
        riota = lax.iota(jnp.int32, LANES)
        zero16 = jnp.zeros((LANES,), jnp.float32)

        for i in range(LANES):
            for j in range(C // 16):
                mb[i, pl.ds(j * 16, 16)] = zero16

        def zero_acc(t, _):
            pltpu.sync_copy(mb, acc.at[pl.ds(sid * ROWS_S + t * 16, 16)])
            return _
        lax.fori_loop(0, ROWS_S // 16, zero_acc, 0)

        if first:
            for i in range(LANES):
                oneb[i, :] = zero16
            plsc.store_scatter(oneb, [riota, jnp.zeros((LANES,), jnp.int32)],
                               jnp.ones((LANES,), jnp.float32))
            for i in range(64):
                zb16[i, :] = zero16

            def zero_acc16(t, _):
                pltpu.sync_copy(zb16, acc16.at[pl.ds(sid * ROWS_S + t * 64, 64)])
                return _
            lax.fori_loop(0, ROWS_S // 64, zero_acc16, 0)
            plsc.subcore_barrier()

            def deg_blk(t, _):
                pltpu.sync_copy(colh.at[pl.ds(sid * EPS + t * BLK, BLK)], colv)

                def deg_step(g, _2):
                    cv = colv[pl.ds(g * LANES, LANES)]
                    pltpu.async_copy(oneb, acc16.at[cv], semq[0], add=True)
                    return _2
                return lax.fori_loop(0, NGB, deg_step, _)
            lax.fori_loop(0, EPS // BLK, deg_blk, 0)

            def deg_drain(g, _):
                pltpu.make_async_copy(oneb, acc16.at[riota], semq[0]).wait()
                return _
            lax.fori_loop(0, EPS // LANES, deg_drain, 0)
            plsc.subcore_barrier()

            def dis_step(g, _):
                pltpu.sync_copy(
                    acc16.at[pl.ds(sid * ROWS_S + g * LANES, LANES)], degs)
                dg = plsc.load_gather(
                    degs, [riota, jnp.zeros((LANES,), jnp.int32)])
                node = sid * ROWS_S + g * LANES + riota
                ok = jnp.logical_and(dg > 0.0, node < N)
                diss[pl.ds(g * LANES, LANES)] = jnp.where(
                    ok, _quake_rsqrt(dg), 0.0)
                return _
            lax.fori_loop(0, ROWS_S // LANES, dis_step, 0)
            pltpu.sync_copy(diss, dis_sh.at[pl.ds(sid * ROWS_S, ROWS_S)])
            plsc.subcore_barrier()
            pltpu.sync_copy(dis_sh, disv)
        else:
            plsc.subcore_barrier()

        def issue(slot, g):
            rv = rowv[pl.ds(g * LANES, LANES)]
            cv = colv[pl.ds(g * LANES, LANES)]
            pltpu.async_copy(qt.at[cv], qb.at[pl.ds(slot * LANES, LANES)],
                             semq[slot])
            pltpu.async_copy(kt.at[rv], kb.at[pl.ds(slot * LANES, LANES)],
                             semk[slot])
            pltpu.async_copy(vt.at[rv], vb.at[pl.ds(slot * LANES, LANES)],
                             semv[slot])

        def drain(slot):
            pltpu.make_async_copy(
                qt.at[riota], qb.at[pl.ds(slot * LANES, LANES)],
                semq[slot]).wait()
            pltpu.make_async_copy(
                kt.at[riota], kb.at[pl.ds(slot * LANES, LANES)],
                semk[slot]).wait()
            pltpu.make_async_copy(
                vt.at[riota], vb.at[pl.ds(slot * LANES, LANES)],
                semv[slot]).wait()

        def compute(slot, g):
            rv = rowv[pl.ds(g * LANES, LANES)]
            cv = colv[pl.ds(g * LANES, LANES)]
            if first:
                nrm = (plsc.load_gather(disv, [rv])
                       * plsc.load_gather(disv, [cv]))
                normv[pl.ds(g * LANES, LANES)] = nrm
            else:
                nrm = normv[pl.ds(g * LANES, LANES)]
            rbase = riota + slot * LANES

            def scores(h):
                hb = h * DH
                s = [None] * L
                for d in range(DH):
                    fv = ((riota + d) & 15) + hb
                    qf = plsc.load_gather(qb, [rbase, fv])
                    for l in range(L):
                        kf = plsc.load_gather(kb, [rbase, fv + l * C])
                        p = qf * kf
                        s[l] = p if s[l] is None else s[l] + p
                return s

            def softmax_w(s):
                if L == 1:
                    m = jnp.maximum(s[0], 0.0)
                    e0 = jnp.exp(s[0] - m)
                    return [nrm * e0 / (e0 + jnp.exp(-m))]
                m = jnp.maximum(jnp.maximum(s[0], s[1]), 0.0)
                e0 = jnp.exp(s[0] - m)
                e1 = jnp.exp(s[1] - m)
                r = nrm / (e0 + e1 + jnp.exp(-m))
                return [e0 * r, e1 * r]

            def vphase(h, w):
                hb = h * DH
                for d in range(DH):
                    fv = ((riota + d) & 15) + hb
                    ov = None
                    for l in range(L):
                        vf = plsc.load_gather(vb, [rbase, fv + l * C])
                        t = w[l] * vf
                        ov = t if ov is None else ov + t
                    plsc.store_scatter(mb, [riota, fv], ov)

            def hstep(h, carry):
                w = softmax_w(list(carry))
                sn = scores(h + 1)
                vphase(h, w)
                return tuple(sn)

            last = lax.fori_loop(0, HEADS - 1, hstep, tuple(scores(0)))
            vphase(HEADS - 1, softmax_w(list(last)))
            pltpu.sync_copy(mb, acc.at[cv], add=True)

        def msg_blk(b, _0):
            pltpu.sync_copy(rowh.at[pl.ds(wid * EPW + b * BLK, BLK)], rowv)
            pltpu.sync_copy(colh.at[pl.ds(wid * EPW + b * BLK, BLK)], colv)
            if not first:
                pltpu.sync_copy(normh.at[pl.ds(wid * EPW + b * BLK, BLK)],
                                normv)
            issue(0, 0)
            issue(1, 1)

            def pair(i, _):
                g0 = 2 * i
                drain(0)
                compute(0, g0)
                issue(0, jnp.minimum(g0 + 2, NGB - 1))
                drain(1)
                compute(1, g0 + 1)
                issue(1, jnp.minimum(g0 + 3, NGB - 1))
                return _
            lax.fori_loop(0, NGB // 2, pair, 0)
            drain(0)
            drain(1)
            if first:
                pltpu.sync_copy(normv,
                                normo.at[pl.ds(wid * EPW + b * BLK, BLK)])
            return _0
        lax.fori_loop(0, NBLK, msg_blk, 0)
        plsc.subcore_barrier()

        pltpu.sync_copy(acc.at[pl.ds(sid * ROWS_S, ROWS_S)],
                        out.at[cid, pl.ds(sid * ROWS_S, ROWS_S)])

    return edge_kernel


def kernel(x, edge_index, W1, b1, Wq0, bq0, Wk0, bk0, Wv0, bv0,
           Wq1, bq1, Wk1, bk1, Wv1, bv1, W2, b2):
    N = x.shape[0]
    E = edge_index.shape[1]
    NC = W2.shape[1]
    NT = ((N + 1 + 255) // 256) * 256
    EP = ((E + N + 3071) // 3072) * 3072

    loops = jnp.arange(N, dtype=jnp.int32)
    npad = EP - E - N
    row = jnp.concatenate([edge_index[0].astype(jnp.int32), loops,
                           jnp.zeros((npad,), jnp.int32)])
    col = jnp.concatenate([edge_index[1].astype(jnp.int32), loops,
                           jnp.full((npad,), N, jnp.int32)])
    xp = jnp.pad(x, ((0, NT - N), (0, 0)))

    scale = 1.0 / np.sqrt(DH)
    Wq0b = _block_diag(Wq0) * scale
    bq0s = (bq0 * scale).reshape(1, C)
    Wq1b = _block_diag(Wq1) * scale
    bq1s = (bq1 * scale).reshape(1, C)
    Wk0b, Wv0b = _block_diag(Wk0), _block_diag(Wv0)
    Wk1b, Wv1b = _block_diag(Wk1), _block_diag(Wv1)
    bk0r, bv0r = bk0.reshape(1, C), bv0.reshape(1, C)
    bk1r, bv1r = bk1.reshape(1, C), bv1.reshape(1, C)
    W2p = jnp.pad(W2, ((0, 0), (0, C - NC)))
    b2p = jnp.concatenate([b2, jnp.full((C - NC,), -1e30, jnp.float32)]).reshape(1, C)

    h, Q0, K0, V0 = _dense1(xp, W1, b1.reshape(1, C), Wq0b, bq0s,
                            Wk0b, bk0r, Wv0b, bv0r)
    magg0, norm = _make_edge_kernel(1, N, NT, EP, True)(Q0, K0, V0, row, col)
    Q1, K1, V1 = _dense2(magg0, h, Wq1b, bq1s, Wk1b, bk1r, Wv1b, bv1r)
    magg1 = _make_edge_kernel(2, N, NT, EP, False)(Q1, K1, V1, row, col, norm)
    outp = _dense3(magg1, W2p, b2p)
    return outp[:N, :NC]


# speedup vs baseline: 1.0298x; 1.0298x over previous
import functools

import jax
import jax.numpy as jnp
import numpy as np
from jax import lax
from jax.experimental import pallas as pl
from jax.experimental.pallas import tpu as pltpu
from jax.experimental.pallas import tpu_sc as plsc

HEADS = 8
DH = 16
C = 128
NCORE = 2
NSUB = 16
NW = NCORE * NSUB
LANES = 16


def _block_diag(W):
    return jax.scipy.linalg.block_diag(*[W[g] for g in range(W.shape[0])])


def _dense1(xp, W1, b1, Wqb, bqb, Wkb, bkb, Wvb, bvb):
    NT = xp.shape[0]
    RB = 1024

    def body(x_ref, w1, b1r, wq, bqr, wk, bkr, wv, bvr, h_ref, q_ref, k_ref, v_ref):
        h = jnp.maximum(
            jnp.dot(x_ref[...], w1[...], preferred_element_type=jnp.float32) + b1r[...], 0.0)
        h_ref[...] = h
        q_ref[...] = jnp.dot(h, wq[...], preferred_element_type=jnp.float32) + bqr[...]
        k_ref[...] = jnp.dot(h, wk[...], preferred_element_type=jnp.float32) + bkr[...]
        v_ref[...] = jnp.dot(h, wv[...], preferred_element_type=jnp.float32) + bvr[...]

    bs_r = pl.BlockSpec((RB, C), lambda i: (i, 0))
    bs_r2 = pl.BlockSpec((RB, 2 * C), lambda i: (i, 0))
    bs_w = pl.BlockSpec((C, C), lambda i: (0, 0))
    bs_b = pl.BlockSpec((1, C), lambda i: (0, 0))
    out = jax.ShapeDtypeStruct((NT, C), jnp.float32)
    return pl.pallas_call(
        body,
        grid=(NT // RB,),
        in_specs=[bs_r, bs_w, bs_b, bs_w, bs_b, bs_w, bs_b, bs_w, bs_b],
        out_specs=[bs_r, bs_r, bs_r, bs_r],
        out_shape=[out, out, out, out],
    )(xp, W1, b1, Wqb, bqb, Wkb, bkb, Wvb, bvb)


def _dense2(magg, h, Wqb, bqb, Wkb, bkb, Wvb, bvb):
    NT = h.shape[0]
    RB = 1024

    def body(m_ref, h_ref, wq, bqr, wk, bkr, wv, bvr, q_ref, k_ref, v_ref):
        m = m_ref[...]
        h0 = h_ref[...]
        h1 = jnp.maximum(m[0] + m[1], 0.0)
        q_ref[...] = jnp.dot(h1, wq[...], preferred_element_type=jnp.float32) + bqr[...]
        k_ref[:, :C] = jnp.dot(h0, wk[...], preferred_element_type=jnp.float32) + bkr[...]
        k_ref[:, C:] = jnp.dot(h1, wk[...], preferred_element_type=jnp.float32) + bkr[...]
        v_ref[:, :C] = jnp.dot(h0, wv[...], preferred_element_type=jnp.float32) + bvr[...]
        v_ref[:, C:] = jnp.dot(h1, wv[...], preferred_element_type=jnp.float32) + bvr[...]

    bs_m = pl.BlockSpec((2, RB, C), lambda i: (0, i, 0))
    bs_r = pl.BlockSpec((RB, C), lambda i: (i, 0))
    bs_r2 = pl.BlockSpec((RB, 2 * C), lambda i: (i, 0))
    bs_w = pl.BlockSpec((C, C), lambda i: (0, 0))
    bs_b = pl.BlockSpec((1, C), lambda i: (0, 0))
    return pl.pallas_call(
        body,
        grid=(NT // RB,),
        in_specs=[bs_m, bs_r, bs_w, bs_b, bs_w, bs_b, bs_w, bs_b],
        out_specs=[bs_r, bs_r2, bs_r2],
        out_shape=[
            jax.ShapeDtypeStruct((NT, C), jnp.float32),
            jax.ShapeDtypeStruct((NT, 2 * C), jnp.float32),
            jax.ShapeDtypeStruct((NT, 2 * C), jnp.float32),
        ],
    )(magg, h, Wqb, bqb, Wkb, bkb, Wvb, bvb)


def _dense3(magg, W2p, b2p):
    NT = magg.shape[1]
    RB = 1024

    def body(m_ref, w2, b2r, o_ref):
        m = m_ref[...]
        x2 = jnp.maximum(m[0] + m[1], 0.0)
        l = jnp.dot(x2, w2[...], preferred_element_type=jnp.float32) + b2r[...]
        mx = jnp.max(l, axis=1, keepdims=True)
        e = jnp.exp(l - mx)
        o_ref[...] = l - mx - jnp.log(jnp.sum(e, axis=1, keepdims=True))

    bs_m = pl.BlockSpec((2, RB, C), lambda i: (0, i, 0))
    bs_r = pl.BlockSpec((RB, C), lambda i: (i, 0))
    bs_w = pl.BlockSpec((C, C), lambda i: (0, 0))
    bs_b = pl.BlockSpec((1, C), lambda i: (0, 0))
    return pl.pallas_call(
        body,
        grid=(NT // RB,),
        in_specs=[bs_m, bs_w, bs_b],
        out_specs=bs_r,
        out_shape=jax.ShapeDtypeStruct((NT, C), jnp.float32),
    )(magg, W2p, b2p)


def _quake_rsqrt(v):
    i = plsc.bitcast(v, jnp.int32)
    i = jnp.int32(0x5F3759DF) - (i >> 1)
    y = plsc.bitcast(i, jnp.float32)
    for _ in range(3):
        y = y * (1.5 - 0.5 * v * y * y)
    return y


def _make_edge_kernel(L, N, NT, EP, first):
    CD = L * C
    EPW = EP // NW
    EPS = EP // NSUB
    NBLK = 3
    BLK = EPW // NBLK
    NGB = BLK // LANES
    ROWS_S = NT // NSUB

    mesh = plsc.VectorSubcoreMesh(core_axis_name="c", subcore_axis_name="s")

    out_type = [jax.ShapeDtypeStruct((NCORE, NT, C), jnp.float32)]
    if first:
        out_type.append(jax.ShapeDtypeStruct((EP,), jnp.float32))

    scratch = [
        pltpu.VMEM((BLK,), jnp.int32),
        pltpu.VMEM((BLK,), jnp.int32),
        pltpu.VMEM((BLK,), jnp.float32),
        pltpu.VMEM((2 * LANES, C), jnp.float32),
        pltpu.VMEM((2 * LANES, CD), jnp.float32),
        pltpu.VMEM((2 * LANES, CD), jnp.float32),
        pltpu.VMEM((LANES, C), jnp.float32),
    ]
    if first:
        scratch += [
            pltpu.VMEM((LANES, LANES), jnp.float32),
            pltpu.VMEM((64, 16), jnp.float32),
            pltpu.VMEM((16, 16), jnp.float32),
            pltpu.VMEM((ROWS_S,), jnp.float32),
            pltpu.VMEM((NT,), jnp.float32),
            pltpu.VMEM_SHARED((NT, 16), jnp.float32),
            pltpu.VMEM_SHARED((NT,), jnp.float32),
        ]
    scratch += [pltpu.VMEM_SHARED((NT, C), jnp.float32)]
    scratch += [pltpu.SemaphoreType.DMA] * 6

    @functools.partial(
        pl.kernel,
        out_type=tuple(out_type) if first else out_type[0],
        mesh=mesh,
        compiler_params=pltpu.CompilerParams(
            needs_layout_passes=False, use_tc_tiling_on_sc=False),
        scratch_types=scratch,
    )
    def edge_kernel(*refs):
        it = iter(refs)
        qt, kt, vt, rowh, colh = (next(it) for _ in range(5))
        normh = None if first else next(it)
        out = next(it)
        normo = next(it) if first else None
        rowv, colv, normv, qb, kb, vb, mb = (next(it) for _ in range(7))
        if first:
            oneb, zb16, degs, diss, disv, acc16, dis_sh = (
                next(it) for _ in range(7))
        acc = next(it)
        sems = [next(it) for _ in range(6)]
        semq, semk, semv = sems[0:2], sems[2:4], sems[4:6]

        cid = lax.axis_index("c")
        sid = lax.axis_index("s")
        wid = sid * NCORE + cid
        riota = lax.iota(jnp.int32, LANES)
        zero16 = jnp.zeros((LANES,), jnp.float32)

        for i in range(LANES):
            for j in range(C // 16):
                mb[i, pl.ds(j * 16, 16)] = zero16

        def zero_acc(t, _):
            pltpu.sync_copy(mb, acc.at[pl.ds(sid * ROWS_S + t * 16, 16)])
            return _
        lax.fori_loop(0, ROWS_S // 16, zero_acc, 0)

        if first:
            for i in range(LANES):
                oneb[i, :] = zero16
            plsc.store_scatter(oneb, [riota, jnp.zeros((LANES,), jnp.int32)],
                               jnp.ones((LANES,), jnp.float32))
            for i in range(64):
                zb16[i, :] = zero16

            def zero_acc16(t, _):
                pltpu.sync_copy(zb16, acc16.at[pl.ds(sid * ROWS_S + t * 64, 64)])
                return _
            lax.fori_loop(0, ROWS_S // 64, zero_acc16, 0)
            plsc.subcore_barrier()

            def deg_blk(t, _):
                pltpu.sync_copy(colh.at[pl.ds(sid * EPS + t * BLK, BLK)], colv)

                def deg_step(g, _2):
                    cv = colv[pl.ds(g * LANES, LANES)]
                    pltpu.async_copy(oneb, acc16.at[cv], semq[0], add=True)
                    return _2
                return lax.fori_loop(0, NGB, deg_step, _)
            lax.fori_loop(0, EPS // BLK, deg_blk, 0)

            def deg_drain(g, _):
                pltpu.make_async_copy(oneb, acc16.at[riota], semq[0]).wait()
                return _
            lax.fori_loop(0, EPS // LANES, deg_drain, 0)
            plsc.subcore_barrier()

            def dis_step(g, _):
                pltpu.sync_copy(
                    acc16.at[pl.ds(sid * ROWS_S + g * LANES, LANES)], degs)
                dg = plsc.load_gather(
                    degs, [riota, jnp.zeros((LANES,), jnp.int32)])
                node = sid * ROWS_S + g * LANES + riota
                ok = jnp.logical_and(dg > 0.0, node < N)
                diss[pl.ds(g * LANES, LANES)] = jnp.where(
                    ok, _quake_rsqrt(dg), 0.0)
                return _
            lax.fori_loop(0, ROWS_S // LANES, dis_step, 0)
            pltpu.sync_copy(diss, dis_sh.at[pl.ds(sid * ROWS_S, ROWS_S)])
            plsc.subcore_barrier()
            pltpu.sync_copy(dis_sh, disv)
        else:
            plsc.subcore_barrier()

        def issue(slot, g):
            rv = rowv[pl.ds(g * LANES, LANES)]
            cv = colv[pl.ds(g * LANES, LANES)]
            pltpu.async_copy(qt.at[cv], qb.at[pl.ds(slot * LANES, LANES)],
                             semq[slot])
            pltpu.async_copy(kt.at[rv], kb.at[pl.ds(slot * LANES, LANES)],
                             semk[slot])
            pltpu.async_copy(vt.at[rv], vb.at[pl.ds(slot * LANES, LANES)],
                             semv[slot])

        def drain(slot):
            pltpu.make_async_copy(
                qt.at[riota], qb.at[pl.ds(slot * LANES, LANES)],
                semq[slot]).wait()
            pltpu.make_async_copy(
                kt.at[riota], kb.at[pl.ds(slot * LANES, LANES)],
                semk[slot]).wait()
            pltpu.make_async_copy(
                vt.at[riota], vb.at[pl.ds(slot * LANES, LANES)],
                semv[slot]).wait()

        def compute(slot, g):
            rv = rowv[pl.ds(g * LANES, LANES)]
            cv = colv[pl.ds(g * LANES, LANES)]
            if first:
                nrm = (plsc.load_gather(disv, [rv])
                       * plsc.load_gather(disv, [cv]))
                normv[pl.ds(g * LANES, LANES)] = nrm
            else:
                nrm = normv[pl.ds(g * LANES, LANES)]
            rbase = riota + slot * LANES

            def head_pair(hh, _3):
                for u in range(2):
                    head(hh * 2 + u)
                return _3

            def head(h):
                hb = h * DH
                s = [None] * L
                for d in range(DH):
                    fv = ((riota + d) & 15) + hb
                    qf = plsc.load_gather(qb, [rbase, fv])
                    for l in range(L):
                        kf = plsc.load_gather(kb, [rbase, fv + l * C])
                        p = qf * kf
                        s[l] = p if s[l] is None else s[l] + p
                if L == 1:
                    m = jnp.maximum(s[0], 0.0)
                    e0 = jnp.exp(s[0] - m)
                    w = [nrm * e0 / (e0 + jnp.exp(-m))]
                else:
                    m = jnp.maximum(jnp.maximum(s[0], s[1]), 0.0)
                    e0 = jnp.exp(s[0] - m)
                    e1 = jnp.exp(s[1] - m)
                    r = nrm / (e0 + e1 + jnp.exp(-m))
                    w = [e0 * r, e1 * r]
                for d in range(DH):
                    fv = ((riota + d) & 15) + hb
                    ov = None
                    for l in range(L):
                        vf = plsc.load_gather(vb, [rbase, fv + l * C])
                        t = w[l] * vf
                        ov = t if ov is None else ov + t
                    plsc.store_scatter(mb, [riota, fv], ov)
            lax.fori_loop(0, HEADS // 2, head_pair, 0)
            pltpu.sync_copy(mb, acc.at[cv], add=True)

        def msg_blk(b, _0):
            pltpu.sync_copy(rowh.at[pl.ds(wid * EPW + b * BLK, BLK)], rowv)
            pltpu.sync_copy(colh.at[pl.ds(wid * EPW + b * BLK, BLK)], colv)
            if not first:
                pltpu.sync_copy(normh.at[pl.ds(wid * EPW + b * BLK, BLK)],
                                normv)
            issue(0, 0)
            issue(1, 1)

            def pair(i, _):
                g0 = 2 * i
                drain(0)
                compute(0, g0)
                issue(0, jnp.minimum(g0 + 2, NGB - 1))
                drain(1)
                compute(1, g0 + 1)
                issue(1, jnp.minimum(g0 + 3, NGB - 1))
                return _
            lax.fori_loop(0, NGB // 2, pair, 0)
            drain(0)
            drain(1)
            if first:
                pltpu.sync_copy(normv,
                                normo.at[pl.ds(wid * EPW + b * BLK, BLK)])
            return _0
        lax.fori_loop(0, NBLK, msg_blk, 0)
        plsc.subcore_barrier()

        pltpu.sync_copy(acc.at[pl.ds(sid * ROWS_S, ROWS_S)],
                        out.at[cid, pl.ds(sid * ROWS_S, ROWS_S)])

    return edge_kernel


def kernel(x, edge_index, W1, b1, Wq0, bq0, Wk0, bk0, Wv0, bv0,
           Wq1, bq1, Wk1, bk1, Wv1, bv1, W2, b2):
    N = x.shape[0]
    E = edge_index.shape[1]
    NC = W2.shape[1]
    NT = ((N + 1 + 255) // 256) * 256
    EP = ((E + N + 3071) // 3072) * 3072

    loops = jnp.arange(N, dtype=jnp.int32)
    npad = EP - E - N
    row = jnp.concatenate([edge_index[0].astype(jnp.int32), loops,
                           jnp.zeros((npad,), jnp.int32)])
    col = jnp.concatenate([edge_index[1].astype(jnp.int32), loops,
                           jnp.full((npad,), N, jnp.int32)])
    xp = jnp.pad(x, ((0, NT - N), (0, 0)))

    scale = 1.0 / np.sqrt(DH)
    Wq0b = _block_diag(Wq0) * scale
    bq0s = (bq0 * scale).reshape(1, C)
    Wq1b = _block_diag(Wq1) * scale
    bq1s = (bq1 * scale).reshape(1, C)
    Wk0b, Wv0b = _block_diag(Wk0), _block_diag(Wv0)
    Wk1b, Wv1b = _block_diag(Wk1), _block_diag(Wv1)
    bk0r, bv0r = bk0.reshape(1, C), bv0.reshape(1, C)
    bk1r, bv1r = bk1.reshape(1, C), bv1.reshape(1, C)
    W2p = jnp.pad(W2, ((0, 0), (0, C - NC)))
    b2p = jnp.concatenate([b2, jnp.full((C - NC,), -1e30, jnp.float32)]).reshape(1, C)

    h, Q0, K0, V0 = _dense1(xp, W1, b1.reshape(1, C), Wq0b, bq0s,
                            Wk0b, bk0r, Wv0b, bv0r)
    magg0, norm = _make_edge_kernel(1, N, NT, EP, True)(Q0, K0, V0, row, col)
    Q1, K1, V1 = _dense2(magg0, h, Wq1b, bq1s, Wk1b, bk1r, Wv1b, bv1r)
    magg1 = _make_edge_kernel(2, N, NT, EP, False)(Q1, K1, V1, row, col, norm)
    outp = _dense3(magg1, W2p, b2p)
    return outp[:N, :NC]
